# baseline (speedup 1.0000x reference)
import jax
import jax.numpy as jnp
from jax.experimental import pallas as pl
from jax.experimental.pallas import tpu as pltpu
from jax.experimental.pallas import tpu_sc as plsc

_G, _T, _D = 8, 16, 128
_N = 10000
_TT = _G * _T
_INNER_PRUNE = 0.01
_CROSS_PRUNE = 0.1


def _dense_body(x_ref, tok_ref, wt_ref, adj_ref, tx_ref):
    x = x_ref[...]
    tok = tok_ref[...]
    wt = wt_ref[...]

    logits = jax.lax.dot_general(wt, x, (((1,), (1,)), ((), ())),
                                 preferred_element_type=jnp.float32)
    best = jnp.full((1, _N), -jnp.inf, dtype=jnp.float32)
    route = jnp.zeros((1, _N), dtype=jnp.int32)
    for g in range(_G):
        lg = logits[g:g + 1, :]
        upd = lg > best
        best = jnp.where(upd, lg, best)
        route = jnp.where(upd, g, route)

    dots = jax.lax.dot_general(tok, x, (((1,), (1,)), ((), ())),
                               preferred_element_type=jnp.float32)
    sim = jax.nn.sigmoid(dots)
    pruned = jnp.where(sim < _CROSS_PRUNE, 0.0, sim)
    gidx = jax.lax.broadcasted_iota(jnp.int32, (_TT, 1), 0) // _T
    cross = jnp.where(gidx == route, pruned, 0.0)
    adj_ref[:, _T:] = cross

    gram = jax.lax.dot_general(tok, tok, (((1,), (1,)), ((), ())),
                               preferred_element_type=jnp.float32)
    gsim = jax.nn.sigmoid(gram)
    gpruned = jnp.where(gsim < _INNER_PRUNE, 0.0, gsim)
    for g in range(_G):
        adj_ref[g * _T:(g + 1) * _T, 0:_T] = (
            gpruned[g * _T:(g + 1) * _T, g * _T:(g + 1) * _T])

    tx_ref[0:_TT, :] = tok
    tx_ref[_TT:, :] = x


def _dense_call(x, tok, wt, interpret=False):
    return pl.pallas_call(
        _dense_body,
        out_shape=[
            jax.ShapeDtypeStruct((_TT, _T + _N), jnp.float32),
            jax.ShapeDtypeStruct((_TT + _N, _D), jnp.float32),
        ],
        interpret=interpret,
    )(x, tok, wt)


def _edge_shift(edge_index):
    rows = edge_index.size // 128
    flat = edge_index.reshape(rows, 128)
    mesh = plsc.VectorSubcoreMesh(core_axis_name="c", subcore_axis_name="s")

    @pl.kernel(out_type=jax.ShapeDtypeStruct(flat.shape, flat.dtype),
               mesh=mesh)
    def _shift_kernel(in_hbm, out_hbm):
        def body(in_vmem, out_vmem):
            @pl.loop(0, 8)
            def _row(r):
                @pl.loop(0, 128, step=16)
                def _col(c):
                    out_vmem.at[pl.ds(r, 1), pl.ds(c, 16)][...] = (
                        in_vmem.at[pl.ds(r, 1), pl.ds(c, 16)][...] + _TT)

        pltpu.emit_pipeline(
            body,
            grid=(rows // 8, 1),
            in_specs=[pl.BlockSpec((8, 128), lambda i, j: (i, j))],
            out_specs=[pl.BlockSpec((8, 128), lambda i, j: (i, j))],
            core_axis_name=("c", "s"),
            dimension_semantics=(pltpu.PARALLEL, pltpu.PARALLEL),
        )(in_hbm, out_hbm)

    return _shift_kernel(flat).reshape(edge_index.shape)


def kernel(x, tokens, pseudo_W, edge_index):
    tok = tokens.reshape(_TT, _D)
    wt = pseudo_W.T
    adj2d, total_x = _dense_call(x, tok, wt)
    adj = adj2d.reshape(_G, _T, _T + _N)
    g_edge_index = _edge_shift(edge_index)
    return adj, total_x, g_edge_index
